# Initial kernel scaffold; baseline (speedup 1.0000x reference)
import jax
import jax.numpy as jnp
from jax.experimental import pallas as pl


def kernel(species, embedding):
    raise NotImplementedError("write your pallas kernel here")


# speedup vs baseline: 5.0630x; 5.0630x over previous
import functools

import jax
import jax.numpy as jnp
from jax import lax
from jax.experimental import pallas as pl
from jax.experimental.pallas import tpu as pltpu
from jax.experimental.pallas import tpu_sc as plsc

ROWS, COLS = 16384, 200
VOCAB, DIM = 100000, 64
B = ROWS * COLS

NC, NS = 2, 16
NW = NC * NS

BPW = B // NW
GATHER = 128
CH = 512
G = CH // GATHER
NCHUNK = BPW // CH
NBUF = 2
NPAIR = NCHUNK // NBUF

assert BPW * NW == B and G * GATHER == CH and NCHUNK * CH == BPW
assert NPAIR * NBUF == NCHUNK


def _emb_body(species_hbm, table_hbm, out_hbm,
              idx_v, rows_v, sem_g0, sem_g1, sem_o0, sem_o1):
    wid = lax.axis_index("s") * NC + lax.axis_index("c")
    base = wid * BPW
    gbase = wid * (BPW // GATHER)
    sem_g = (sem_g0, sem_g1)
    sem_o = (sem_o0, sem_o1)

    def load_and_fire(c, b):
        pltpu.sync_copy(species_hbm.at[pl.ds(gbase + c * G, G)], idx_v.at[b])
        for j in range(G):
            pltpu.async_copy(
                table_hbm.at[idx_v.at[b, j]],
                rows_v.at[b, pl.ds(j * GATHER, GATHER)],
                sem_g[b],
            )

    def drain_gathers(b):
        for j in range(G):
            pltpu.make_async_copy(
                table_hbm.at[idx_v.at[b, j]],
                rows_v.at[b, pl.ds(j * GATHER, GATHER)],
                sem_g[b],
            ).wait()

    for b in range(NBUF):
        load_and_fire(b, b)

    def pair_body(p, _):
        for b in range(NBUF):
            c = p * NBUF + b
            drain_gathers(b)
            store = pltpu.async_copy(
                rows_v.at[b], out_hbm.at[pl.ds(base + c * CH, CH)], sem_o[b])
            store.wait()
            load_and_fire(c + NBUF, b)
        return 0

    lax.fori_loop(0, NPAIR - 1, pair_body, 0)

    for b in range(NBUF):
        c = (NPAIR - 1) * NBUF + b
        drain_gathers(b)
        pltpu.async_copy(
            rows_v.at[b], out_hbm.at[pl.ds(base + c * CH, CH)], sem_o[b]
        ).wait()


@jax.jit
def _embed(species_flat, table):
    mesh = plsc.VectorSubcoreMesh(
        core_axis_name="c", subcore_axis_name="s",
        num_cores=NC, num_subcores=NS)
    run = pl.kernel(
        _emb_body,
        out_type=jax.ShapeDtypeStruct((B, DIM), jnp.float32),
        mesh=mesh,
        scratch_types=[
            pltpu.VMEM((NBUF, G, GATHER), jnp.int32),
            pltpu.VMEM((NBUF, CH, DIM), jnp.float32),
            pltpu.SemaphoreType.DMA,
            pltpu.SemaphoreType.DMA,
            pltpu.SemaphoreType.DMA,
            pltpu.SemaphoreType.DMA,
        ],
        compiler_params=pltpu.CompilerParams(use_tc_tiling_on_sc=False),
    )
    return run(species_flat, table)


def kernel(species, embedding):
    species_flat = species.reshape(B // GATHER, GATHER).astype(jnp.int32)
    out = _embed(species_flat, embedding)
    return out.reshape(ROWS, COLS, DIM)
